# Initial kernel scaffold; baseline (speedup 1.0000x reference)
import jax
import jax.numpy as jnp
from jax.experimental import pallas as pl


def kernel(x, edge_index, W1, b1, W2, b2, W3, b3):
    raise NotImplementedError("write your pallas kernel here")


# speedup vs baseline: 7.1396x; 7.1396x over previous
import functools

import jax
import jax.numpy as jnp
from jax import lax
from jax.experimental import pallas as pl
from jax.experimental.pallas import tpu as pltpu
from jax.experimental.pallas import tpu_sc as plsc

N = 10000
E = 320000
D_IN = 128
H = 128

NC = 2
NS = 16
CB = 128
CHUNKS = 80
EPT = CB * CHUNKS
EP = NC * NS * EPT
NP = 10240
RPT = NP // NS

_mesh = plsc.VectorSubcoreMesh(core_axis_name="c", subcore_axis_name="s")


@functools.partial(
    pl.kernel,
    out_type=jax.ShapeDtypeStruct((NC, NP, H), jnp.float32),
    mesh=_mesh,
    scratch_types=[
        pltpu.VMEM((CHUNKS, CB), jnp.int32),
        pltpu.VMEM((CB, H), jnp.float32),
        pltpu.VMEM((CB, H), jnp.float32),
        pltpu.VMEM_SHARED((NP, H), jnp.float32),
    ],
)
def _sc_degree(dstp, ones, zeros, deg_out, didx, ov, r0, acc):
    c = lax.axis_index("c")
    s = lax.axis_index("s")
    pltpu.sync_copy(dstp.at[c, s], didx)
    pltpu.sync_copy(ones, ov)
    pltpu.sync_copy(zeros, r0)
    base = s * RPT
    for t in range(RPT // CB):
        pltpu.sync_copy(r0, acc.at[pl.ds(base + t * CB, CB)])
    plsc.subcore_barrier()

    def body(j, carry):
        pltpu.sync_copy(ov, acc.at[didx.at[j]], add=True)
        return carry

    lax.fori_loop(0, CHUNKS, body, 0)
    plsc.subcore_barrier()
    for t in range(RPT // CB):
        pltpu.sync_copy(acc.at[pl.ds(base + t * CB, CB)], r0)
        pltpu.sync_copy(r0, deg_out.at[c, pl.ds(base + t * CB, CB)])


HALF = CHUNKS // 2


@functools.partial(
    pl.kernel,
    out_type=jax.ShapeDtypeStruct((NC, NP, H), jnp.float32),
    mesh=_mesh,
    scratch_types=[
        pltpu.VMEM((HALF, CB), jnp.int32),
        pltpu.VMEM((HALF, CB), jnp.int32),
        pltpu.VMEM((CB, H), jnp.float32),
        pltpu.VMEM((CB, H), jnp.float32),
        pltpu.VMEM_SHARED((NP, H), jnp.float32),
        pltpu.SemaphoreType.DMA,
        pltpu.SemaphoreType.DMA,
    ],
)
def _sc_scatter(hn, srcp, dstp, zeros, acc_out, sidx, didx, r0, r1, acc, s0, s1):
    c = lax.axis_index("c")
    s = lax.axis_index("s")
    base = s * RPT
    pltpu.sync_copy(zeros, r0)
    for t in range(RPT // CB):
        pltpu.sync_copy(r0, acc.at[pl.ds(base + t * CB, CB)])
    plsc.subcore_barrier()

    for h in range(CHUNKS // HALF):
        pltpu.sync_copy(srcp.at[c, s, h], sidx)
        pltpu.sync_copy(dstp.at[c, s, h], didx)

        def body(jj, carry):
            j = 2 * jj
            d0 = pltpu.async_copy(hn.at[sidx.at[j]], r0, s0)
            d1 = pltpu.async_copy(hn.at[sidx.at[j + 1]], r1, s1)
            d0.wait()
            pltpu.sync_copy(r0, acc.at[didx.at[j]], add=True)
            d1.wait()
            pltpu.sync_copy(r1, acc.at[didx.at[j + 1]], add=True)
            return carry

        lax.fori_loop(0, HALF // 2, body, 0)
    plsc.subcore_barrier()
    for t in range(RPT // CB):
        pltpu.sync_copy(acc.at[pl.ds(base + t * CB, CB)], r0)
        pltpu.sync_copy(r0, acc_out.at[c, pl.ds(base + t * CB, CB)])


BR = 400


def _tc_first_body(x_ref, w_ref, d0_ref, d1_ref, hn_ref, dinv_ref):
    dv = lax.rsqrt(d0_ref[...] + d1_ref[...] + 1.0)
    hn_ref[...] = jnp.dot(x_ref[...], w_ref[...],
                          preferred_element_type=jnp.float32) * dv
    dinv_ref[...] = dv


def _tc_mid_body(a0_ref, a1_ref, hnp_ref, dinv_ref, b_ref, w_ref, hn_ref):
    dv = dinv_ref[...]
    a = jnp.maximum(dv * (a0_ref[...] + a1_ref[...] + hnp_ref[...]) + b_ref[...],
                    0.0)
    hn_ref[...] = jnp.dot(a, w_ref[...], preferred_element_type=jnp.float32) * dv


def _tc_final_body(a0_ref, a1_ref, hnp_ref, dinv_ref, b_ref, out_ref):
    out_ref[...] = (dinv_ref[...] * (a0_ref[...] + a1_ref[...] + hnp_ref[...])
                    + b_ref[...])


def _row_spec(width):
    return pl.BlockSpec((BR, width), lambda i: (i, 0))


def _full_spec(rows, cols):
    return pl.BlockSpec((rows, cols), lambda i: (0, 0))


def _tc_first(x, w, d0, d1):
    return pl.pallas_call(
        _tc_first_body,
        grid=(N // BR,),
        in_specs=[_row_spec(D_IN), _full_spec(D_IN, H), _row_spec(1), _row_spec(1)],
        out_specs=[_row_spec(H), _row_spec(1)],
        out_shape=[
            jax.ShapeDtypeStruct((N, H), jnp.float32),
            jax.ShapeDtypeStruct((N, 1), jnp.float32),
        ],
    )(x, w, d0, d1)


def _tc_mid(a0, a1, hnp, dinv, b, w):
    return pl.pallas_call(
        _tc_mid_body,
        grid=(N // BR,),
        in_specs=[_row_spec(H), _row_spec(H), _row_spec(H), _row_spec(1),
                  _full_spec(1, H), _full_spec(H, H)],
        out_specs=_row_spec(H),
        out_shape=jax.ShapeDtypeStruct((N, H), jnp.float32),
    )(a0, a1, hnp, dinv, b, w)


def _tc_final(a0, a1, hnp, dinv, b):
    return pl.pallas_call(
        _tc_final_body,
        grid=(N // BR,),
        in_specs=[_row_spec(H), _row_spec(H), _row_spec(H), _row_spec(1),
                  _full_spec(1, H)],
        out_specs=_row_spec(H),
        out_shape=jax.ShapeDtypeStruct((N, H), jnp.float32),
    )(a0, a1, hnp, dinv, b)


def kernel(x, edge_index, W1, b1, W2, b2, W3, b3):
    pad = EP - E
    srcp = jnp.concatenate(
        [edge_index[0], jnp.zeros((pad,), jnp.int32)]
    ).reshape(NC, NS, CHUNKS // HALF, HALF, CB)
    dstp4 = jnp.concatenate(
        [edge_index[1], jnp.full((pad,), N, jnp.int32)]
    ).reshape(NC, NS, CHUNKS, CB)
    dstp = dstp4.reshape(NC, NS, CHUNKS // HALF, HALF, CB)
    ones128 = jnp.ones((CB, H), jnp.float32)
    zrows = jnp.zeros((CB, H), jnp.float32)

    degs = _sc_degree(dstp4, ones128, zrows)
    d0 = degs[0, :N, 0:1]
    d1 = degs[1, :N, 0:1]

    hn1, dinv = _tc_first(x, W1, d0, d1)
    accp = _sc_scatter(hn1, srcp, dstp, zrows)
    hn2 = _tc_mid(accp[0, :N], accp[1, :N], hn1, dinv, b1.reshape(1, H), W2)
    accp = _sc_scatter(hn2, srcp, dstp, zrows)
    hn3 = _tc_mid(accp[0, :N], accp[1, :N], hn2, dinv, b2.reshape(1, H), W3)
    accp = _sc_scatter(hn3, srcp, dstp, zrows)
    return _tc_final(accp[0, :N], accp[1, :N], hn3, dinv, b3.reshape(1, H))


# speedup vs baseline: 7.8527x; 1.0999x over previous
import functools

import jax
import jax.numpy as jnp
from jax import lax
from jax.experimental import pallas as pl
from jax.experimental.pallas import tpu as pltpu
from jax.experimental.pallas import tpu_sc as plsc

N = 10000
E = 320000
D_IN = 128
H = 128

NC = 2
NS = 16
CB = 128
CHUNKS = 80
EPT = CB * CHUNKS
EP = NC * NS * EPT
NP = 10240
RPT = NP // NS

_mesh = plsc.VectorSubcoreMesh(core_axis_name="c", subcore_axis_name="s")


@functools.partial(
    pl.kernel,
    out_type=jax.ShapeDtypeStruct((NC, NP, H), jnp.float32),
    mesh=_mesh,
    scratch_types=[
        pltpu.VMEM((CHUNKS, CB), jnp.int32),
        pltpu.VMEM((CB, H), jnp.float32),
        pltpu.VMEM((CB, H), jnp.float32),
        pltpu.VMEM_SHARED((NP, H), jnp.float32),
    ],
)
def _sc_degree(dstp, ones, zeros, deg_out, didx, ov, r0, acc):
    c = lax.axis_index("c")
    s = lax.axis_index("s")
    pltpu.sync_copy(dstp.at[c, s], didx)
    pltpu.sync_copy(ones, ov)
    pltpu.sync_copy(zeros, r0)
    base = s * RPT
    for t in range(RPT // CB):
        pltpu.sync_copy(r0, acc.at[pl.ds(base + t * CB, CB)])
    plsc.subcore_barrier()

    def body(j, carry):
        pltpu.sync_copy(ov, acc.at[didx.at[j]], add=True)
        return carry

    lax.fori_loop(0, CHUNKS, body, 0)
    plsc.subcore_barrier()
    for t in range(RPT // CB):
        pltpu.sync_copy(acc.at[pl.ds(base + t * CB, CB)], r0)
        pltpu.sync_copy(r0, deg_out.at[c, pl.ds(base + t * CB, CB)])


HALF = CHUNKS // 2


@functools.partial(
    pl.kernel,
    out_type=jax.ShapeDtypeStruct((NC, NP, H), jnp.float32),
    mesh=_mesh,
    scratch_types=[
        pltpu.VMEM((HALF, CB), jnp.int32),
        pltpu.VMEM((HALF, CB), jnp.int32),
        pltpu.VMEM((CB, H), jnp.float32),
        pltpu.VMEM((CB, H), jnp.float32),
        pltpu.VMEM_SHARED((NP, H), jnp.float32),
        pltpu.SemaphoreType.DMA,
        pltpu.SemaphoreType.DMA,
    ],
)
def _sc_scatter(hn, srcp, dstp, zeros, acc_out, sidx, didx, r0, r1, acc, s0, s1):
    c = lax.axis_index("c")
    s = lax.axis_index("s")
    base = s * RPT
    pltpu.sync_copy(zeros, r0)
    for t in range(RPT // CB):
        pltpu.sync_copy(r0, acc.at[pl.ds(base + t * CB, CB)])
    plsc.subcore_barrier()

    for h in range(CHUNKS // HALF):
        pltpu.sync_copy(srcp.at[c, s, h], sidx)
        pltpu.sync_copy(dstp.at[c, s, h], didx)
        pltpu.async_copy(hn.at[sidx.at[0]], r0, s0)

        def body(jj, carry):
            j = 2 * jj
            pltpu.async_copy(hn.at[sidx.at[j + 1]], r1, s1)
            pltpu.make_async_copy(hn.at[sidx.at[j]], r0, s0).wait()
            pltpu.sync_copy(r0, acc.at[didx.at[j]], add=True)

            @pl.when(jj < HALF // 2 - 1)
            def _():
                pltpu.async_copy(hn.at[sidx.at[j + 2]], r0, s0)

            pltpu.make_async_copy(hn.at[sidx.at[j + 1]], r1, s1).wait()
            pltpu.sync_copy(r1, acc.at[didx.at[j + 1]], add=True)
            return carry

        lax.fori_loop(0, HALF // 2, body, 0)
    plsc.subcore_barrier()
    for t in range(RPT // CB):
        pltpu.sync_copy(acc.at[pl.ds(base + t * CB, CB)], r0)
        pltpu.sync_copy(r0, acc_out.at[c, pl.ds(base + t * CB, CB)])


BR = 400


def _tc_first_body(x_ref, w_ref, d0_ref, d1_ref, hn_ref, dinv_ref):
    dv = lax.rsqrt(d0_ref[...] + d1_ref[...] + 1.0)
    hn_ref[...] = jnp.dot(x_ref[...], w_ref[...],
                          preferred_element_type=jnp.float32) * dv
    dinv_ref[...] = dv


def _tc_mid_body(a0_ref, a1_ref, hnp_ref, dinv_ref, b_ref, w_ref, hn_ref):
    dv = dinv_ref[...]
    a = jnp.maximum(dv * (a0_ref[...] + a1_ref[...] + hnp_ref[...]) + b_ref[...],
                    0.0)
    hn_ref[...] = jnp.dot(a, w_ref[...], preferred_element_type=jnp.float32) * dv


def _tc_final_body(a0_ref, a1_ref, hnp_ref, dinv_ref, b_ref, out_ref):
    out_ref[...] = (dinv_ref[...] * (a0_ref[...] + a1_ref[...] + hnp_ref[...])
                    + b_ref[...])


def _row_spec(width):
    return pl.BlockSpec((BR, width), lambda i: (i, 0))


def _full_spec(rows, cols):
    return pl.BlockSpec((rows, cols), lambda i: (0, 0))


def _tc_first(x, w, d0, d1):
    return pl.pallas_call(
        _tc_first_body,
        grid=(N // BR,),
        in_specs=[_row_spec(D_IN), _full_spec(D_IN, H), _row_spec(1), _row_spec(1)],
        out_specs=[_row_spec(H), _row_spec(1)],
        out_shape=[
            jax.ShapeDtypeStruct((N, H), jnp.float32),
            jax.ShapeDtypeStruct((N, 1), jnp.float32),
        ],
    )(x, w, d0, d1)


def _tc_mid(a0, a1, hnp, dinv, b, w):
    return pl.pallas_call(
        _tc_mid_body,
        grid=(N // BR,),
        in_specs=[_row_spec(H), _row_spec(H), _row_spec(H), _row_spec(1),
                  _full_spec(1, H), _full_spec(H, H)],
        out_specs=_row_spec(H),
        out_shape=jax.ShapeDtypeStruct((N, H), jnp.float32),
    )(a0, a1, hnp, dinv, b, w)


def _tc_final(a0, a1, hnp, dinv, b):
    return pl.pallas_call(
        _tc_final_body,
        grid=(N // BR,),
        in_specs=[_row_spec(H), _row_spec(H), _row_spec(H), _row_spec(1),
                  _full_spec(1, H)],
        out_specs=_row_spec(H),
        out_shape=jax.ShapeDtypeStruct((N, H), jnp.float32),
    )(a0, a1, hnp, dinv, b)


def kernel(x, edge_index, W1, b1, W2, b2, W3, b3):
    pad = EP - E
    srcp = jnp.concatenate(
        [edge_index[0], jnp.zeros((pad,), jnp.int32)]
    ).reshape(NC, NS, CHUNKS // HALF, HALF, CB)
    dstp4 = jnp.concatenate(
        [edge_index[1], jnp.full((pad,), N, jnp.int32)]
    ).reshape(NC, NS, CHUNKS, CB)
    dstp = dstp4.reshape(NC, NS, CHUNKS // HALF, HALF, CB)
    ones128 = jnp.ones((CB, H), jnp.float32)
    zrows = jnp.zeros((CB, H), jnp.float32)

    degs = _sc_degree(dstp4, ones128, zrows)
    d0 = degs[0, :N, 0:1]
    d1 = degs[1, :N, 0:1]

    hn1, dinv = _tc_first(x, W1, d0, d1)
    accp = _sc_scatter(hn1, srcp, dstp, zrows)
    hn2 = _tc_mid(accp[0, :N], accp[1, :N], hn1, dinv, b1.reshape(1, H), W2)
    accp = _sc_scatter(hn2, srcp, dstp, zrows)
    hn3 = _tc_mid(accp[0, :N], accp[1, :N], hn2, dinv, b2.reshape(1, H), W3)
    accp = _sc_scatter(hn3, srcp, dstp, zrows)
    return _tc_final(accp[0, :N], accp[1, :N], hn3, dinv, b3.reshape(1, H))
